# baseline (speedup 1.0000x reference)
import functools

import jax
import jax.numpy as jnp
from jax import lax
from jax.experimental import pallas as pl
from jax.experimental.pallas import tpu as pltpu
from jax.experimental.pallas import tpu_sc as plsc

IN_DIM = 256
HIDDEN = 512
NVN = 64
TOPK = 5
NG = 8
N0 = 10000
BLK = 2000
ROWS = NG * NVN
CHUNK = 2000
NCH = N0 // CHUNK
NW = 32
TPW = ROWS // NW


def _emb_kernel(x_ref, we_ref, be_ref, amat_ref, batch_ref,
                aff_ref, s_ref):
    i = pl.program_id(0)
    h = jnp.dot(x_ref[...], we_ref[...],
                preferred_element_type=jnp.float32) + be_ref[...]
    aff_ref[...] = jnp.dot(h, amat_ref[...],
                           preferred_element_type=jnp.float32)
    g8 = jax.lax.broadcasted_iota(jnp.int32, (BLK, NG), 1)
    onehot = (batch_ref[...] == g8).astype(jnp.float32)
    part = jax.lax.dot_general(
        onehot, h, (((0,), (0,)), ((), ())),
        preferred_element_type=jnp.float32,
        precision=jax.lax.Precision.HIGHEST)

    @pl.when(i == 0)
    def _():
        s_ref[...] = jnp.zeros_like(s_ref)

    s_ref[...] = s_ref[...] + part


def _topk_kernel(aff_ref, sb_ref, denom_ref, base_ref, csum_ref,
                 coef_ref, idxo_ref):
    neg_inf = jnp.float32(-jnp.inf)
    iota_c = jax.lax.broadcasted_iota(jnp.int32, (CHUNK, NVN), 0)

    bounds = [(sb_ref[g], sb_ref[g + 1]) for g in range(NG)]
    fin = []
    for g in range(NG):
        s, e = bounds[g]

        def body(c, carry, _s=s, _e=e):
            m_run, i_run = carry
            a = aff_ref[pl.ds(c * CHUNK, CHUNK), :]
            gidx = iota_c + c * CHUNK
            am = jnp.where((gidx >= _s) & (gidx < _e), a, neg_inf)
            lm, li = [], []
            for r in range(TOPK):
                cm = jnp.max(am, axis=0, keepdims=True)
                ci = jnp.min(
                    jnp.where((am == cm) & (cm > neg_inf), gidx, N0),
                    axis=0, keepdims=True)
                lm.append(cm)
                li.append(ci)
                if r < TOPK - 1:
                    am = jnp.where(gidx == ci, neg_inf, am)
            cv = jnp.concatenate(lm + [m_run], 0)
            civ = jnp.concatenate(li + [i_run], 0)
            nm, ni = [], []
            for r in range(TOPK):
                mm = jnp.max(cv, axis=0, keepdims=True)
                nn = jnp.min(
                    jnp.where((cv == mm) & (mm > neg_inf), civ, N0),
                    axis=0, keepdims=True)
                nm.append(mm)
                ni.append(nn)
                if r < TOPK - 1:
                    cv = jnp.where(civ == nn, neg_inf, cv)
            return (jnp.concatenate(nm, 0), jnp.concatenate(ni, 0))

        c0 = s // CHUNK
        ce = jnp.maximum((e + CHUNK - 1) // CHUNK, c0)
        init = (jnp.full((TOPK, NVN), neg_inf, jnp.float32),
                jnp.full((TOPK, NVN), N0, jnp.int32))
        fin.append(jax.lax.fori_loop(c0, ce, body, init))

    vals = [jnp.concatenate([fin[g][0][r:r + 1] for g in range(NG)], 0)
            for r in range(TOPK)]
    idxs = [jnp.concatenate([fin[g][1][r:r + 1] for g in range(NG)], 0)
            for r in range(TOPK)]

    M = jnp.maximum(vals[0], 0.0)
    base = jnp.exp(-M)
    cnt = jnp.concatenate(
        [jnp.full((1, NVN), 1.0, jnp.float32) * (e - s).astype(jnp.float32)
         for (s, e) in bounds], 0)
    denom = cnt * base
    csum = jnp.zeros((NG, NVN), jnp.float32)
    for r in range(TOPK):
        valid = vals[r] > neg_inf
        wv = jnp.where(valid, jnp.exp(vals[r] - M), 0.0)
        coef = jnp.where(valid, wv - base, 0.0)
        denom = denom + coef
        csum = csum + coef
        coef_ref[r * NG:(r + 1) * NG, :] = coef
        idxo_ref[r * NG:(r + 1) * NG, :] = idxs[r]
    denom_ref[...] = denom
    base_ref[...] = base
    csum_ref[...] = csum


def _sc_corr_kernel(idx_hbm, coef_hbm, x_hbm, out_hbm,
                    idx_v, coef_v, rows_v, acc_v, sem):
    w = lax.axis_index("s") * 2 + lax.axis_index("c")
    nidx = TPW * TOPK
    pltpu.sync_copy(idx_hbm.at[pl.ds(w * nidx, nidx)], idx_v)
    pltpu.sync_copy(coef_hbm.at[pl.ds(w * nidx * 16, nidx * 16)], coef_v)
    pltpu.async_copy(x_hbm.at[idx_v], rows_v, sem).wait()

    def task(t, carry):
        for c in range(IN_DIM // 16):
            acc = jnp.zeros((16,), jnp.float32)
            for j in range(TOPK):
                k = t * TOPK + j
                cf = coef_v[pl.ds(k * 16, 16)]
                row = rows_v[k, pl.ds(c * 16, 16)]
                acc = acc + cf * row
            acc_v[pl.ds(t * IN_DIM + c * 16, 16)] = acc
        return carry

    lax.fori_loop(0, TPW, task, 0)
    pltpu.sync_copy(acc_v, out_hbm.at[pl.ds(w * TPW * IN_DIM, TPW * IN_DIM)])


def _head_kernel(s_ref, xc_ref, base_ref, den_ref, csum_ref,
                 we_ref, be_ref,
                 wv1_ref, bv1_ref, wv2_ref, bv2_ref,
                 wm1_ref, bm1_ref, wm2_ref, bm2_ref, out_ref):
    seg = s_ref[...]
    base = base_ref[...]
    den = den_ref[...]
    csum = csum_ref[...]
    corr = jnp.dot(xc_ref[...], we_ref[...],
                   preferred_element_type=jnp.float32)
    corr3 = corr.reshape(NG, NVN, HIDDEN) \
        + csum[:, :, None] * be_ref[...][None, :, :]
    num = base[:, :, None] * seg[:, None, :] + corr3
    vn3 = num * (1.0 / den)[:, :, None]
    vn = vn3.reshape(ROWS, HIDDEN)
    z = jnp.maximum(jnp.dot(vn, wv1_ref[...],
                            preferred_element_type=jnp.float32)
                    + bv1_ref[...], 0.0)
    z = jnp.dot(z, wv2_ref[...],
                preferred_element_type=jnp.float32) + bv2_ref[...]
    row = jax.lax.broadcasted_iota(jnp.int32, (NG, ROWS), 0)
    col = jax.lax.broadcasted_iota(jnp.int32, (NG, ROWS), 1)
    pool = jnp.where(col // NVN == row, jnp.float32(1.0 / NVN), 0.0)
    gf = jnp.dot(pool, z, preferred_element_type=jnp.float32)
    y = jnp.maximum(jnp.dot(gf, wm1_ref[...],
                            preferred_element_type=jnp.float32)
                    + bm1_ref[...], 0.0)
    out_ref[...] = jnp.dot(y, wm2_ref[...],
                           preferred_element_type=jnp.float32) + bm2_ref[...]


def kernel(x, edge_index, batch, W_emb, b_emb, affinity_scores,
           Wv1, bv1, Wv2, bv2, Wm1, bm1, Wm2, bm2):
    bi = batch.astype(jnp.int32)
    bp = bi[:, None]
    amat = affinity_scores[0]

    nblk = N0 // BLK
    aff, seg_sum = pl.pallas_call(
        _emb_kernel,
        grid=(nblk,),
        in_specs=[
            pl.BlockSpec((BLK, IN_DIM), lambda i: (i, 0)),
            pl.BlockSpec((IN_DIM, HIDDEN), lambda i: (0, 0)),
            pl.BlockSpec((1, HIDDEN), lambda i: (0, 0)),
            pl.BlockSpec((HIDDEN, NVN), lambda i: (0, 0)),
            pl.BlockSpec((BLK, 1), lambda i: (i, 0)),
        ],
        out_specs=[
            pl.BlockSpec((BLK, NVN), lambda i: (i, 0)),
            pl.BlockSpec((NG, HIDDEN), lambda i: (0, 0)),
        ],
        out_shape=[
            jax.ShapeDtypeStruct((N0, NVN), jnp.float32),
            jax.ShapeDtypeStruct((NG, HIDDEN), jnp.float32),
        ],
    )(x, W_emb, b_emb[None, :], amat, bp)

    sb = jnp.searchsorted(bi, jnp.arange(NG + 1, dtype=jnp.int32)
                          ).astype(jnp.int32)
    sb = jnp.pad(sb, (0, 16 - (NG + 1)))

    denom, base, csum, coef, idxo = pl.pallas_call(
        _topk_kernel,
        in_specs=[
            pl.BlockSpec((N0, NVN), lambda: (0, 0)),
            pl.BlockSpec(memory_space=pltpu.SMEM),
        ],
        out_specs=[
            pl.BlockSpec((NG, NVN), lambda: (0, 0)),
            pl.BlockSpec((NG, NVN), lambda: (0, 0)),
            pl.BlockSpec((NG, NVN), lambda: (0, 0)),
            pl.BlockSpec((TOPK * NG, NVN), lambda: (0, 0)),
            pl.BlockSpec((TOPK * NG, NVN), lambda: (0, 0)),
        ],
        out_shape=[
            jax.ShapeDtypeStruct((NG, NVN), jnp.float32),
            jax.ShapeDtypeStruct((NG, NVN), jnp.float32),
            jax.ShapeDtypeStruct((NG, NVN), jnp.float32),
            jax.ShapeDtypeStruct((TOPK * NG, NVN), jnp.float32),
            jax.ShapeDtypeStruct((TOPK * NG, NVN), jnp.int32),
        ],
    )(aff, sb)

    idx_t = jnp.minimum(idxo.reshape(TOPK, NG, NVN), N0 - 1)
    idx_flat = jnp.transpose(idx_t, (1, 2, 0)).reshape(ROWS * TOPK)
    coef_t = jnp.transpose(coef.reshape(TOPK, NG, NVN), (1, 2, 0))
    coef_rep = jnp.repeat(coef_t.reshape(ROWS * TOPK), 16)

    sc_corr = functools.partial(
        pl.kernel,
        out_type=jax.ShapeDtypeStruct((ROWS * IN_DIM,), jnp.float32),
        mesh=plsc.VectorSubcoreMesh(core_axis_name="c", subcore_axis_name="s"),
        scratch_types=[
            pltpu.VMEM((TPW * TOPK,), jnp.int32),
            pltpu.VMEM((TPW * TOPK * 16,), jnp.float32),
            pltpu.VMEM((TPW * TOPK, IN_DIM), jnp.float32),
            pltpu.VMEM((TPW * IN_DIM,), jnp.float32),
            pltpu.SemaphoreType.DMA,
        ],
    )(_sc_corr_kernel)
    xc = sc_corr(idx_flat, coef_rep, x).reshape(ROWS, IN_DIM)

    out = pl.pallas_call(
        _head_kernel,
        in_specs=[
            pl.BlockSpec((NG, HIDDEN), lambda: (0, 0)),
            pl.BlockSpec((ROWS, IN_DIM), lambda: (0, 0)),
            pl.BlockSpec((NG, NVN), lambda: (0, 0)),
            pl.BlockSpec((NG, NVN), lambda: (0, 0)),
            pl.BlockSpec((NG, NVN), lambda: (0, 0)),
            pl.BlockSpec((IN_DIM, HIDDEN), lambda: (0, 0)),
            pl.BlockSpec((1, HIDDEN), lambda: (0, 0)),
            pl.BlockSpec((HIDDEN, HIDDEN), lambda: (0, 0)),
            pl.BlockSpec((1, HIDDEN), lambda: (0, 0)),
            pl.BlockSpec((HIDDEN, HIDDEN), lambda: (0, 0)),
            pl.BlockSpec((1, HIDDEN), lambda: (0, 0)),
            pl.BlockSpec((HIDDEN, HIDDEN), lambda: (0, 0)),
            pl.BlockSpec((1, HIDDEN), lambda: (0, 0)),
            pl.BlockSpec((HIDDEN, 128), lambda: (0, 0)),
            pl.BlockSpec((1, 128), lambda: (0, 0)),
        ],
        out_specs=pl.BlockSpec((NG, 128), lambda: (0, 0)),
        out_shape=jax.ShapeDtypeStruct((NG, 128), jnp.float32),
    )(seg_sum, xc, base, denom, csum, W_emb, b_emb[None, :],
      Wv1, bv1[None, :], Wv2, bv2[None, :],
      Wm1, bm1[None, :], Wm2, bm2[None, :])
    return out


# speedup vs baseline: 1.1138x; 1.1138x over previous
import functools

import jax
import jax.numpy as jnp
from jax import lax
from jax.experimental import pallas as pl
from jax.experimental.pallas import tpu as pltpu
from jax.experimental.pallas import tpu_sc as plsc

IN_DIM = 256
HIDDEN = 512
NVN = 64
TOPK = 5
NG = 8
N0 = 10000
BLK = 1000
ROWS = NG * NVN
CHUNK = 1000
NCH = N0 // CHUNK
NW = 32
TPW = ROWS // NW


def _fused_kernel(x_ref, we_ref, be_ref, amat_ref, batch_ref, sb_ref,
                  s_ref, denom_ref, base_ref, csum_ref, coef_ref, idxo_ref,
                  aff_ref):
    i = pl.program_id(0)
    h = jnp.dot(x_ref[...], we_ref[...],
                preferred_element_type=jnp.float32) + be_ref[...]
    aff_ref[pl.ds(i * BLK, BLK), :] = jnp.dot(
        h, amat_ref[...], preferred_element_type=jnp.float32)
    g8 = jax.lax.broadcasted_iota(jnp.int32, (BLK, NG), 1)
    onehot = (batch_ref[...] == g8).astype(jnp.float32)
    part = jax.lax.dot_general(
        onehot, h, (((0,), (0,)), ((), ())),
        preferred_element_type=jnp.float32,
        precision=jax.lax.Precision.HIGHEST)

    @pl.when(i == 0)
    def _():
        s_ref[...] = jnp.zeros_like(s_ref)

    s_ref[...] = s_ref[...] + part

    @pl.when(i == N0 // BLK - 1)
    def _():
        _topk_body(aff_ref, sb_ref, denom_ref, base_ref, csum_ref,
                   coef_ref, idxo_ref)


def _topk_body(aff_ref, sb_ref, denom_ref, base_ref, csum_ref,
               coef_ref, idxo_ref):
    neg_inf = jnp.float32(-jnp.inf)
    iota_c = jax.lax.broadcasted_iota(jnp.int32, (CHUNK, NVN), 0)

    bounds = [(sb_ref[g], sb_ref[g + 1]) for g in range(NG)]
    fin = []
    for g in range(NG):
        s, e = bounds[g]

        def body(c, carry, _s=s, _e=e):
            m_run, i_run = carry
            a = aff_ref[pl.ds(c * CHUNK, CHUNK), :]
            gidx = iota_c + c * CHUNK
            am = jnp.where((gidx >= _s) & (gidx < _e), a, neg_inf)
            lm, li = [], []
            for r in range(TOPK):
                cm = jnp.max(am, axis=0, keepdims=True)
                ci = jnp.min(
                    jnp.where((am == cm) & (cm > neg_inf), gidx, N0),
                    axis=0, keepdims=True)
                lm.append(cm)
                li.append(ci)
                if r < TOPK - 1:
                    am = jnp.where(gidx == ci, neg_inf, am)
            cv = jnp.concatenate(lm + [m_run], 0)
            civ = jnp.concatenate(li + [i_run], 0)
            nm, ni = [], []
            for r in range(TOPK):
                mm = jnp.max(cv, axis=0, keepdims=True)
                nn = jnp.min(
                    jnp.where((cv == mm) & (mm > neg_inf), civ, N0),
                    axis=0, keepdims=True)
                nm.append(mm)
                ni.append(nn)
                if r < TOPK - 1:
                    cv = jnp.where(civ == nn, neg_inf, cv)
            return (jnp.concatenate(nm, 0), jnp.concatenate(ni, 0))

        c0 = s // CHUNK
        ce = jnp.maximum((e + CHUNK - 1) // CHUNK, c0)
        init = (jnp.full((TOPK, NVN), neg_inf, jnp.float32),
                jnp.full((TOPK, NVN), N0, jnp.int32))
        fin.append(jax.lax.fori_loop(c0, ce, body, init))

    vals = [jnp.concatenate([fin[g][0][r:r + 1] for g in range(NG)], 0)
            for r in range(TOPK)]
    idxs = [jnp.concatenate([fin[g][1][r:r + 1] for g in range(NG)], 0)
            for r in range(TOPK)]

    M = jnp.maximum(vals[0], 0.0)
    base = jnp.exp(-M)
    cnt = jnp.concatenate(
        [jnp.full((1, NVN), 1.0, jnp.float32) * (e - s).astype(jnp.float32)
         for (s, e) in bounds], 0)
    denom = cnt * base
    csum = jnp.zeros((NG, NVN), jnp.float32)
    for r in range(TOPK):
        valid = vals[r] > neg_inf
        wv = jnp.where(valid, jnp.exp(vals[r] - M), 0.0)
        coef = jnp.where(valid, wv - base, 0.0)
        denom = denom + coef
        csum = csum + coef
        coef_ref[r * NG:(r + 1) * NG, :] = coef
        idxo_ref[r * NG:(r + 1) * NG, :] = idxs[r]
    denom_ref[...] = denom
    base_ref[...] = base
    csum_ref[...] = csum


def _sc_corr_kernel(idx_hbm, coef_hbm, x_hbm, out_hbm,
                    idx_v, coef_v, rows_v, acc_v, sem):
    w = lax.axis_index("s") * 2 + lax.axis_index("c")
    nidx = TPW * TOPK
    pltpu.sync_copy(idx_hbm.at[pl.ds(w * nidx, nidx)], idx_v)
    pltpu.sync_copy(coef_hbm.at[pl.ds(w * nidx * 16, nidx * 16)], coef_v)
    pltpu.async_copy(x_hbm.at[idx_v], rows_v, sem).wait()

    def task(t, carry):
        for c in range(IN_DIM // 16):
            acc = jnp.zeros((16,), jnp.float32)
            for j in range(TOPK):
                k = t * TOPK + j
                cf = coef_v[pl.ds(k * 16, 16)]
                row = rows_v[k, pl.ds(c * 16, 16)]
                acc = acc + cf * row
            acc_v[pl.ds(t * IN_DIM + c * 16, 16)] = acc
        return carry

    lax.fori_loop(0, TPW, task, 0)
    pltpu.sync_copy(acc_v, out_hbm.at[pl.ds(w * TPW * IN_DIM, TPW * IN_DIM)])


def _head_kernel(s_ref, xc_ref, base_ref, den_ref, csum_ref,
                 we_ref, be_ref,
                 wv1_ref, bv1_ref, wv2_ref, bv2_ref,
                 wm1_ref, bm1_ref, wm2_ref, bm2_ref, out_ref):
    seg = s_ref[...]
    base = base_ref[...]
    den = den_ref[...]
    csum = csum_ref[...]
    corr = jnp.dot(xc_ref[...], we_ref[...],
                   preferred_element_type=jnp.float32)
    corr3 = corr.reshape(NG, NVN, HIDDEN) \
        + csum[:, :, None] * be_ref[...][None, :, :]
    num = base[:, :, None] * seg[:, None, :] + corr3
    vn3 = num * (1.0 / den)[:, :, None]
    vn = vn3.reshape(ROWS, HIDDEN)
    z = jnp.maximum(jnp.dot(vn, wv1_ref[...],
                            preferred_element_type=jnp.float32)
                    + bv1_ref[...], 0.0)
    z = jnp.dot(z, wv2_ref[...],
                preferred_element_type=jnp.float32) + bv2_ref[...]
    row = jax.lax.broadcasted_iota(jnp.int32, (NG, ROWS), 0)
    col = jax.lax.broadcasted_iota(jnp.int32, (NG, ROWS), 1)
    pool = jnp.where(col // NVN == row, jnp.float32(1.0 / NVN), 0.0)
    gf = jnp.dot(pool, z, preferred_element_type=jnp.float32)
    y = jnp.maximum(jnp.dot(gf, wm1_ref[...],
                            preferred_element_type=jnp.float32)
                    + bm1_ref[...], 0.0)
    out_ref[...] = jnp.dot(y, wm2_ref[...],
                           preferred_element_type=jnp.float32) + bm2_ref[...]


def kernel(x, edge_index, batch, W_emb, b_emb, affinity_scores,
           Wv1, bv1, Wv2, bv2, Wm1, bm1, Wm2, bm2):
    bi = batch.astype(jnp.int32)
    bp = bi[:, None]
    amat = affinity_scores[0]

    sb = jnp.searchsorted(bi, jnp.arange(NG + 1, dtype=jnp.int32)
                          ).astype(jnp.int32)
    sb = jnp.pad(sb, (0, 16 - (NG + 1)))

    nblk = N0 // BLK
    seg_sum, denom, base, csum, coef, idxo = pl.pallas_call(
        _fused_kernel,
        grid=(nblk,),
        in_specs=[
            pl.BlockSpec((BLK, IN_DIM), lambda i: (i, 0)),
            pl.BlockSpec((IN_DIM, HIDDEN), lambda i: (0, 0)),
            pl.BlockSpec((1, HIDDEN), lambda i: (0, 0)),
            pl.BlockSpec((HIDDEN, NVN), lambda i: (0, 0)),
            pl.BlockSpec((BLK, 1), lambda i: (i, 0)),
            pl.BlockSpec(memory_space=pltpu.SMEM),
        ],
        out_specs=[
            pl.BlockSpec((NG, HIDDEN), lambda i: (0, 0)),
            pl.BlockSpec((NG, NVN), lambda i: (0, 0)),
            pl.BlockSpec((NG, NVN), lambda i: (0, 0)),
            pl.BlockSpec((NG, NVN), lambda i: (0, 0)),
            pl.BlockSpec((TOPK * NG, NVN), lambda i: (0, 0)),
            pl.BlockSpec((TOPK * NG, NVN), lambda i: (0, 0)),
        ],
        out_shape=[
            jax.ShapeDtypeStruct((NG, HIDDEN), jnp.float32),
            jax.ShapeDtypeStruct((NG, NVN), jnp.float32),
            jax.ShapeDtypeStruct((NG, NVN), jnp.float32),
            jax.ShapeDtypeStruct((NG, NVN), jnp.float32),
            jax.ShapeDtypeStruct((TOPK * NG, NVN), jnp.float32),
            jax.ShapeDtypeStruct((TOPK * NG, NVN), jnp.int32),
        ],
        scratch_shapes=[pltpu.VMEM((N0, NVN), jnp.float32)],
    )(x, W_emb, b_emb[None, :], amat, bp, sb)

    idx_t = jnp.minimum(idxo.reshape(TOPK, NG, NVN), N0 - 1)
    idx_flat = jnp.transpose(idx_t, (1, 2, 0)).reshape(ROWS * TOPK)
    coef_t = jnp.transpose(coef.reshape(TOPK, NG, NVN), (1, 2, 0))
    coef_rep = jnp.repeat(coef_t.reshape(ROWS * TOPK), 16)

    sc_corr = functools.partial(
        pl.kernel,
        out_type=jax.ShapeDtypeStruct((ROWS * IN_DIM,), jnp.float32),
        mesh=plsc.VectorSubcoreMesh(core_axis_name="c", subcore_axis_name="s"),
        scratch_types=[
            pltpu.VMEM((TPW * TOPK,), jnp.int32),
            pltpu.VMEM((TPW * TOPK * 16,), jnp.float32),
            pltpu.VMEM((TPW * TOPK, IN_DIM), jnp.float32),
            pltpu.VMEM((TPW * IN_DIM,), jnp.float32),
            pltpu.SemaphoreType.DMA,
        ],
    )(_sc_corr_kernel)
    xc = sc_corr(idx_flat, coef_rep, x).reshape(ROWS, IN_DIM)

    out = pl.pallas_call(
        _head_kernel,
        in_specs=[
            pl.BlockSpec((NG, HIDDEN), lambda: (0, 0)),
            pl.BlockSpec((ROWS, IN_DIM), lambda: (0, 0)),
            pl.BlockSpec((NG, NVN), lambda: (0, 0)),
            pl.BlockSpec((NG, NVN), lambda: (0, 0)),
            pl.BlockSpec((NG, NVN), lambda: (0, 0)),
            pl.BlockSpec((IN_DIM, HIDDEN), lambda: (0, 0)),
            pl.BlockSpec((1, HIDDEN), lambda: (0, 0)),
            pl.BlockSpec((HIDDEN, HIDDEN), lambda: (0, 0)),
            pl.BlockSpec((1, HIDDEN), lambda: (0, 0)),
            pl.BlockSpec((HIDDEN, HIDDEN), lambda: (0, 0)),
            pl.BlockSpec((1, HIDDEN), lambda: (0, 0)),
            pl.BlockSpec((HIDDEN, HIDDEN), lambda: (0, 0)),
            pl.BlockSpec((1, HIDDEN), lambda: (0, 0)),
            pl.BlockSpec((HIDDEN, 128), lambda: (0, 0)),
            pl.BlockSpec((1, 128), lambda: (0, 0)),
        ],
        out_specs=pl.BlockSpec((NG, 128), lambda: (0, 0)),
        out_shape=jax.ShapeDtypeStruct((NG, 128), jnp.float32),
    )(seg_sum, xc, base, denom, csum, W_emb, b_emb[None, :],
      Wv1, bv1[None, :], Wv2, bv2[None, :],
      Wm1, bm1[None, :], Wm2, bm2[None, :])
    return out
